# Initial kernel scaffold; baseline (speedup 1.0000x reference)
import functools

import jax
import jax.numpy as jnp
from jax import lax
from jax.experimental import pallas as pl
from jax.experimental.pallas import tpu as pltpu
from jax.experimental.pallas import tpu_sc as plsc

N = 10000
E = 320000
F = 128

NC = 2
NS = 16
NW = NC * NS
L = 16

CH = 128
NCHUNK = 79
EPW = CH * NCHUNK
E_PAD = EPW * NW
ACC_ROWS = 10240
RPT_Z = ACC_ROWS // NS
RPT_O = N // NS

_MESH = plsc.VectorSubcoreMesh(core_axis_name="c", subcore_axis_name="s")


@functools.partial(
    pl.kernel,
    mesh=_MESH,
    out_type=jax.ShapeDtypeStruct((NC, N, F), jnp.float32),
    scratch_types=[
        pltpu.VMEM((NCHUNK, CH), jnp.int32),
        pltpu.VMEM((NCHUNK, CH), jnp.int32),
        pltpu.VMEM((CH, F), jnp.float32),
        pltpu.VMEM((16, F), jnp.float32),
        pltpu.VMEM_SHARED((ACC_ROWS, F), jnp.float32),
        pltpu.SemaphoreType.DMA,
    ],
)
def _segsum(h_hbm, col_hbm, row_hbm, out_hbm, colb, rowb, rows, zb, acc, sem):
    c = lax.axis_index("c")
    s = lax.axis_index("s")
    w = c * NS + s

    zv = jnp.zeros((L,), jnp.float32)
    for i in range(16):
        for j in range(F // L):
            zb[i, pl.ds(j * L, L)] = zv

    def zloop(k, carry):
        pltpu.sync_copy(zb, acc.at[pl.ds(s * RPT_Z + k * 16, 16), :])
        return carry

    lax.fori_loop(0, RPT_Z // 16, zloop, 0)

    pltpu.sync_copy(col_hbm.at[w], colb)
    pltpu.sync_copy(row_hbm.at[w], rowb)
    plsc.subcore_barrier()

    def body(g, carry):
        pltpu.async_copy(h_hbm.at[colb.at[g]], rows, sem).wait()
        pltpu.sync_copy(rows, acc.at[rowb.at[g]], add=True)
        return carry

    lax.fori_loop(0, NCHUNK, body, 0)

    plsc.subcore_barrier()
    pltpu.sync_copy(acc.at[pl.ds(s * RPT_O, RPT_O), :],
                    out_hbm.at[c, pl.ds(s * RPT_O, RPT_O), :])


DEG_EPT = E // NS
DEG_BLK = 2000
DEG_RPT = ACC_ROWS // NS


@functools.partial(
    pl.kernel,
    mesh=_MESH,
    out_type=jax.ShapeDtypeStruct((ACC_ROWS,), jnp.float32),
    scratch_types=[
        pltpu.VMEM((DEG_BLK,), jnp.int32),
        pltpu.VMEM((ACC_ROWS,), jnp.float32),
        pltpu.VMEM((NS, DEG_RPT), jnp.float32),
        pltpu.VMEM_SHARED((NS, ACC_ROWS), jnp.float32),
    ],
)
def _degree(row_hbm, out_hbm, rbuf, hist, red, sbuf):
    c = lax.axis_index("c")
    s = lax.axis_index("s")

    zv = jnp.zeros((L,), jnp.float32)

    def z(k, carry):
        hist[pl.ds(k * L, L)] = zv
        return carry

    lax.fori_loop(0, ACC_ROWS // L, z, 0)

    ones = jnp.ones((L,), jnp.float32)
    for blk in range(DEG_EPT // DEG_BLK):
        pltpu.sync_copy(
            row_hbm.at[pl.ds(s * DEG_EPT + blk * DEG_BLK, DEG_BLK)], rbuf)

        def b(v, carry):
            idx = rbuf[pl.ds(v * L, L)]
            plsc.addupdate_scatter(hist, [idx], ones)
            return carry

        lax.fori_loop(0, DEG_BLK // L, b, 0)

    pltpu.sync_copy(hist, sbuf.at[s])
    plsc.subcore_barrier()

    @pl.when(c == 0)
    def _():
        pltpu.sync_copy(sbuf.at[:, pl.ds(s * DEG_RPT, DEG_RPT)], red)

        def sv(v, carry):
            t = red[0, pl.ds(v * L, L)]
            for tt in range(1, NS):
                t = t + red[tt, pl.ds(v * L, L)]
            red[0, pl.ds(v * L, L)] = t
            return carry

        lax.fori_loop(0, DEG_RPT // L, sv, 0)
        pltpu.sync_copy(red.at[0], out_hbm.at[pl.ds(s * DEG_RPT, DEG_RPT)])


BLK = 1000


def _dense_mid(part, h, x, deg, wt, wb, b, wci, wco, bc, o):
    sup = (part[0] + part[1]) / deg[...]
    hv = h[...]
    xv = x[...]
    out = (jnp.dot(hv, wt[...], preferred_element_type=jnp.float32)
           + jnp.dot(sup, wb[...], preferred_element_type=jnp.float32)
           + b[...])
    z = jax.nn.sigmoid(jnp.dot(xv, wci[...], preferred_element_type=jnp.float32)
                       + jnp.dot(out, wco[...], preferred_element_type=jnp.float32)
                       + bc[...])
    g = z * out + (1.0 - z) * xv
    o[...] = jnp.maximum(g, 0.0)


def _dense_fin(part, h, x, deg, wt, wb, b, wci, wco, bc, wfc, bfc, o):
    sup = (part[0] + part[1]) / deg[...]
    hv = h[...]
    xv = x[...]
    out = (jnp.dot(hv, wt[...], preferred_element_type=jnp.float32)
           + jnp.dot(sup, wb[...], preferred_element_type=jnp.float32)
           + b[...])
    z = jax.nn.sigmoid(jnp.dot(xv, wci[...], preferred_element_type=jnp.float32)
                       + jnp.dot(out, wco[...], preferred_element_type=jnp.float32)
                       + bc[...])
    g = z * out + (1.0 - z) * xv
    o[...] = jnp.dot(g, wfc[...], preferred_element_type=jnp.float32) + bfc[...]


def _row_spec():
    return pl.BlockSpec((BLK, F), lambda i: (i, 0))


def _full_spec(shape):
    nd = len(shape)
    return pl.BlockSpec(shape, lambda i: (0,) * nd)


def _dense_call(final, part, h, x, deg2, wt, wb, b, wci, wco, bc, *fc):
    in_specs = [
        pl.BlockSpec((NC, BLK, F), lambda i: (0, i, 0)),
        _row_spec(),
        _row_spec(),
        pl.BlockSpec((BLK, 1), lambda i: (i, 0)),
        _full_spec((F, F)), _full_spec((F, F)),
        _full_spec((1, F)),
        _full_spec((F, F)), _full_spec((F, F)),
        _full_spec((1, F)),
    ]
    args = [part, h, x, deg2, wt, wb, b, wci, wco, bc]
    if final:
        in_specs += [_full_spec((F, F)), _full_spec((1, F))]
        args += list(fc)
    return pl.pallas_call(
        _dense_fin if final else _dense_mid,
        grid=(N // BLK,),
        in_specs=in_specs,
        out_specs=_row_spec(),
        out_shape=jax.ShapeDtypeStruct((N, F), jnp.float32),
    )(*args)


def kernel(x, edge_index, W0, b0, W1, b1, W2, b2, Wci, bci, Wco, bco, Wfc, bfc):
    row = edge_index[0]
    col = edge_index[1]

    pad = E_PAD - E
    pad_i = jnp.arange(pad, dtype=jnp.int32)
    row3 = jnp.concatenate([row, N + pad_i % (ACC_ROWS - N)]).reshape(
        NW, NCHUNK, CH)
    col3 = jnp.concatenate([col, pad_i % 64]).reshape(NW, NCHUNK, CH)

    deg2 = _degree(row)[:N].reshape(N, 1)

    bc = (bci + bco).reshape(1, F)
    bfc2 = bfc.reshape(1, F)

    part = _segsum(x, col3, row3)
    h1 = _dense_call(False, part, x, x, deg2, W0[:F], W0[F:], b0,
                     Wci, Wco, bc)
    part = _segsum(h1, col3, row3)
    h2 = _dense_call(False, part, h1, x, deg2, W1[:F], W1[F:], b1,
                     Wci, Wco, bc)
    part = _segsum(h2, col3, row3)
    return _dense_call(True, part, h2, x, deg2, W2[:F], W2[F:], b2,
                       Wci, Wco, bc, Wfc, bfc2)


# speedup vs baseline: 3.9684x; 3.9684x over previous
import functools

import jax
import jax.numpy as jnp
from jax import lax
from jax.experimental import pallas as pl
from jax.experimental.pallas import tpu as pltpu
from jax.experimental.pallas import tpu_sc as plsc

N = 10000
E = 320000
F = 128

NC = 2
NS = 16
NW = NC * NS
L = 16
DW = 16

CH = 64
GRP = 16
NGRP = 10
NCHUNK = GRP * NGRP
EPW = CH * NCHUNK
E_PAD = EPW * NW
ACC_ROWS = 10240
RPT_Z = ACC_ROWS // NS


ZCH = RPT_Z // CH


def _make_segsum_body(with_deg):
    def body_fn(*refs):
        if with_deg:
            (h_hbm, col_hbm, row_hbm, out_hbm, dout_hbm,
             colb, rowb, rows, idxb, ob, zdb, acc, dacc, sem) = refs
        else:
            (h_hbm, col_hbm, row_hbm, out_hbm, dout_hbm,
             colb, rowb, rows, idxb, acc, sem) = refs
        c = lax.axis_index("c")
        s = lax.axis_index("s")
        w = c * NS + s

        zv = jnp.zeros((L,), jnp.float32)

        def zrow(i, carry):
            for j in range(F // L):
                rows[i, pl.ds(j * L, L)] = zv
            return carry
        lax.fori_loop(0, CH, zrow, 0)
        if with_deg:
            ov = jnp.ones((L,), jnp.float32)
            for i in range(CH):
                ob[i, :] = ov
                zdb[i, :] = zv

        iv = lax.iota(jnp.int32, L)

        def fill_idx(base):
            for j in range(CH // L):
                idxb[pl.ds(j * L, L)] = iv + (base + j * L)

        def zloop(k, carry):
            fill_idx(s * RPT_Z + k * CH)
            pltpu.sync_copy(rows, acc.at[idxb])
            if with_deg:
                pltpu.sync_copy(zdb, dacc.at[idxb])
            return carry

        lax.fori_loop(0, ZCH, zloop, 0)
        plsc.subcore_barrier()

        def body(g, carry):
            pltpu.sync_copy(col_hbm.at[w, g], colb)
            pltpu.sync_copy(row_hbm.at[w, g], rowb)
            pltpu.async_copy(h_hbm.at[colb], rows, sem).wait()
            pltpu.sync_copy(rows, acc.at[rowb], add=True)
            if with_deg:
                pltpu.sync_copy(ob, dacc.at[rowb], add=True)
            return carry

        lax.fori_loop(0, NCHUNK, body, 0)

        plsc.subcore_barrier()

        def oloop(k, carry):
            base = s * RPT_Z + k * CH
            fill_idx(base)
            pltpu.async_copy(acc.at[idxb], rows, sem).wait()
            pltpu.sync_copy(rows, out_hbm.at[c, pl.ds(base, CH), :])
            if with_deg:
                pltpu.async_copy(dacc.at[idxb], zdb, sem).wait()
                pltpu.sync_copy(zdb, dout_hbm.at[c, pl.ds(base, CH), :])
            return carry

        lax.fori_loop(0, ZCH, oloop, 0)
    return body_fn


@functools.cache
def _sc_kernels():
    mesh = plsc.VectorSubcoreMesh(core_axis_name="c", subcore_axis_name="s")

    def make(with_deg):
        deg_scratch = [
            pltpu.VMEM((CH, DW), jnp.float32),
            pltpu.VMEM((CH, DW), jnp.float32),
        ] if with_deg else []
        deg_accum = [
            pltpu.VMEM_SHARED((ACC_ROWS, DW), jnp.float32),
        ] if with_deg else []
        dout_rows = ACC_ROWS if with_deg else 16
        return pl.kernel(
            _make_segsum_body(with_deg),
            mesh=mesh,
            out_type=(jax.ShapeDtypeStruct((NC, ACC_ROWS, F), jnp.float32),
                      jax.ShapeDtypeStruct((NC, dout_rows, DW), jnp.float32)),
            scratch_types=[
                pltpu.VMEM((CH,), jnp.int32),
                pltpu.VMEM((CH,), jnp.int32),
                pltpu.VMEM((CH, F), jnp.float32),
                pltpu.VMEM((CH,), jnp.int32),
                *deg_scratch,
                pltpu.VMEM_SHARED((ACC_ROWS, F), jnp.float32),
                *deg_accum,
                pltpu.SemaphoreType.DMA,
            ],
        )

    return make(True), make(False)


BLK = 1000


def _gated(part, h, x, deg, wt, wb, b, wci, wco, bc):
    sup = (part[0] + part[1]) / (deg[0, :, 0:1] + deg[1, :, 0:1])
    hv = h[...]
    xv = x[...]
    out = (jnp.dot(hv, wt[...], preferred_element_type=jnp.float32)
           + jnp.dot(sup, wb[...], preferred_element_type=jnp.float32)
           + b[...])
    z = jax.nn.sigmoid(jnp.dot(xv, wci[...], preferred_element_type=jnp.float32)
                       + jnp.dot(out, wco[...], preferred_element_type=jnp.float32)
                       + bc[...])
    return z * out + (1.0 - z) * xv


def _dense_mid(part, h, x, deg, wt, wb, b, wci, wco, bc, o):
    o[...] = jnp.maximum(_gated(part, h, x, deg, wt, wb, b, wci, wco, bc), 0.0)


def _dense_fin(part, h, x, deg, wt, wb, b, wci, wco, bc, wfc, bfc, o):
    g = _gated(part, h, x, deg, wt, wb, b, wci, wco, bc)
    o[...] = jnp.dot(g, wfc[...], preferred_element_type=jnp.float32) + bfc[...]


def _row_spec():
    return pl.BlockSpec((BLK, F), lambda i: (i, 0))


def _full_spec(shape):
    nd = len(shape)
    return pl.BlockSpec(shape, lambda i: (0,) * nd)


def _dense_call(final, part, h, x, degp, wt, wb, b, wci, wco, bc, *fc):
    in_specs = [
        pl.BlockSpec((NC, BLK, F), lambda i: (0, i, 0)),
        _row_spec(),
        _row_spec(),
        pl.BlockSpec((NC, BLK, DW), lambda i: (0, i, 0)),
        _full_spec((F, F)), _full_spec((F, F)),
        _full_spec((1, F)),
        _full_spec((F, F)), _full_spec((F, F)),
        _full_spec((1, F)),
    ]
    args = [part, h, x, degp, wt, wb, b, wci, wco, bc]
    if final:
        in_specs += [_full_spec((F, F)), _full_spec((1, F))]
        args += list(fc)
    return pl.pallas_call(
        _dense_fin if final else _dense_mid,
        grid=(N // BLK,),
        in_specs=in_specs,
        out_specs=_row_spec(),
        out_shape=jax.ShapeDtypeStruct((N, F), jnp.float32),
    )(*args)


def kernel(x, edge_index, W0, b0, W1, b1, W2, b2, Wci, bci, Wco, bco, Wfc, bfc):
    row = edge_index[0]
    col = edge_index[1]

    pad = E_PAD - E
    pad_i = jnp.arange(pad, dtype=jnp.int32)
    row3 = jnp.concatenate([row, N + pad_i % (ACC_ROWS - N)]).reshape(
        NW, NCHUNK, CH)
    col3 = jnp.concatenate([col, pad_i % 64]).reshape(NW, NCHUNK, CH)

    segsum_deg, segsum = _sc_kernels()

    bc = (bci + bco).reshape(1, F)
    bfc2 = bfc.reshape(1, F)

    part, degp = segsum_deg(x, col3, row3)
    h1 = _dense_call(False, part, x, x, degp, W0[:F], W0[F:], b0,
                     Wci, Wco, bc)
    part, _ = segsum(h1, col3, row3)
    h2 = _dense_call(False, part, h1, x, degp, W1[:F], W1[F:], b1,
                     Wci, Wco, bc)
    part, _ = segsum(h2, col3, row3)
    return _dense_call(True, part, h2, x, degp, W2[:F], W2[F:], b2,
                       Wci, Wco, bc, Wfc, bfc2)


# speedup vs baseline: 8.3944x; 2.1153x over previous
import functools

import jax
import jax.numpy as jnp
from jax import lax
from jax.experimental import pallas as pl
from jax.experimental.pallas import tpu as pltpu
from jax.experimental.pallas import tpu_sc as plsc

N = 10000
E = 320000
F = 128

NC = 2
NS = 16
NW = NC * NS
L = 16
DW = 16

CH = 64
GCH = 8
NGRP = 20
NCHUNK = GCH * NGRP
EPW = CH * NCHUNK
E_PAD = EPW * NW
ACC_ROWS = 10240
RPT_Z = ACC_ROWS // NS


ZCH = RPT_Z // CH


def _make_segsum_body(with_deg):
    def body_fn(*refs):
        if with_deg:
            (h_hbm, col_hbm, row_hbm, out_hbm, dout_hbm,
             colb, rowb, rows0, rows1, idxb, ob, zdb, acc, dacc,
             gsem0, gsem1, ssem0, ssem1, sem) = refs
        else:
            (h_hbm, col_hbm, row_hbm, out_hbm, dout_hbm,
             colb, rowb, rows0, rows1, idxb, acc,
             gsem0, gsem1, ssem0, ssem1, sem) = refs
        rowsb = (rows0, rows1)
        gsem = (gsem0, gsem1)
        ssem = (ssem0, ssem1)
        rows = rows0
        c = lax.axis_index("c")
        s = lax.axis_index("s")
        w = c * NS + s

        zv = jnp.zeros((L,), jnp.float32)

        def zrow(i, carry):
            for j in range(F // L):
                rows[i, pl.ds(j * L, L)] = zv
            return carry
        lax.fori_loop(0, CH, zrow, 0)
        if with_deg:
            ov = jnp.ones((L,), jnp.float32)
            for i in range(CH):
                ob[i, :] = ov
                zdb[i, :] = zv

        iv = lax.iota(jnp.int32, L)

        def fill_idx(base):
            for j in range(CH // L):
                idxb[pl.ds(j * L, L)] = iv + (base + j * L)

        def zloop(k, carry):
            fill_idx(s * RPT_Z + k * CH)
            pltpu.sync_copy(rows, acc.at[idxb])
            if with_deg:
                pltpu.sync_copy(zdb, dacc.at[idxb])
            return carry

        lax.fori_loop(0, ZCH, zloop, 0)
        plsc.subcore_barrier()

        pltpu.sync_copy(col_hbm.at[w, pl.ds(0, GCH)], colb)
        pltpu.sync_copy(row_hbm.at[w, pl.ds(0, GCH)], rowb)
        pltpu.async_copy(h_hbm.at[colb.at[0]], rows0, gsem0)

        def grp_loop(grp, carry):
            for j in range(GCH):
                b = j & 1
                nb = 1 - b
                if j < GCH - 1:
                    if j > 0:
                        pltpu.make_async_copy(rowsb[nb],
                                              acc.at[rowb.at[j]],
                                              ssem[nb]).wait()
                    pltpu.async_copy(h_hbm.at[colb.at[j + 1]], rowsb[nb],
                                     gsem[nb])
                    pltpu.make_async_copy(h_hbm.at[colb.at[j]], rowsb[b],
                                          gsem[b]).wait()
                    pltpu.async_copy(rowsb[b], acc.at[rowb.at[j]], ssem[b],
                                     add=True)
                    if with_deg:
                        pltpu.sync_copy(ob, dacc.at[rowb.at[j]], add=True)
                else:
                    pltpu.make_async_copy(rowsb[nb], acc.at[rowb.at[j]],
                                          ssem[nb]).wait()
                    pltpu.make_async_copy(h_hbm.at[colb.at[j]], rowsb[b],
                                          gsem[b]).wait()
                    pltpu.sync_copy(rowsb[b], acc.at[rowb.at[j]], add=True)
                    if with_deg:
                        pltpu.sync_copy(ob, dacc.at[rowb.at[j]], add=True)

                    @pl.when(grp + 1 < NGRP)
                    def _():
                        pltpu.sync_copy(
                            col_hbm.at[w, pl.ds((grp + 1) * GCH, GCH)], colb)
                        pltpu.sync_copy(
                            row_hbm.at[w, pl.ds((grp + 1) * GCH, GCH)], rowb)
                        pltpu.async_copy(h_hbm.at[colb.at[0]], rowsb[nb],
                                         gsem[nb])
            return carry

        lax.fori_loop(0, NGRP, grp_loop, 0)

        plsc.subcore_barrier()

        def oloop(k, carry):
            base = s * RPT_Z + k * CH
            fill_idx(base)
            pltpu.async_copy(acc.at[idxb], rows, sem).wait()
            pltpu.sync_copy(rows, out_hbm.at[c, pl.ds(base, CH), :])
            if with_deg:
                pltpu.async_copy(dacc.at[idxb], zdb, sem).wait()
                pltpu.sync_copy(zdb, dout_hbm.at[c, pl.ds(base, CH), :])
            return carry

        lax.fori_loop(0, ZCH, oloop, 0)
    return body_fn


@functools.cache
def _sc_kernels():
    mesh = plsc.VectorSubcoreMesh(core_axis_name="c", subcore_axis_name="s")

    def make(with_deg):
        deg_scratch = [
            pltpu.VMEM((CH, DW), jnp.float32),
            pltpu.VMEM((CH, DW), jnp.float32),
        ] if with_deg else []
        deg_accum = [
            pltpu.VMEM_SHARED((ACC_ROWS, DW), jnp.float32),
        ] if with_deg else []
        dout_rows = ACC_ROWS if with_deg else 16
        return pl.kernel(
            _make_segsum_body(with_deg),
            mesh=mesh,
            out_type=(jax.ShapeDtypeStruct((NC, ACC_ROWS, F), jnp.float32),
                      jax.ShapeDtypeStruct((NC, dout_rows, DW), jnp.float32)),
            scratch_types=[
                pltpu.VMEM((GCH, CH), jnp.int32),
                pltpu.VMEM((GCH, CH), jnp.int32),
                pltpu.VMEM((CH, F), jnp.float32),
                pltpu.VMEM((CH, F), jnp.float32),
                pltpu.VMEM((CH,), jnp.int32),
                *deg_scratch,
                pltpu.VMEM_SHARED((ACC_ROWS, F), jnp.float32),
                *deg_accum,
                pltpu.SemaphoreType.DMA,
                pltpu.SemaphoreType.DMA,
                pltpu.SemaphoreType.DMA,
                pltpu.SemaphoreType.DMA,
                pltpu.SemaphoreType.DMA,
            ],
        )

    return make(True), make(False)


BLK = 1000


def _gated(part, h, x, deg, wt, wb, b, wci, wco, bc):
    sup = (part[0] + part[1]) / (deg[0, :, 0:1] + deg[1, :, 0:1])
    hv = h[...]
    xv = x[...]
    out = (jnp.dot(hv, wt[...], preferred_element_type=jnp.float32)
           + jnp.dot(sup, wb[...], preferred_element_type=jnp.float32)
           + b[...])
    z = jax.nn.sigmoid(jnp.dot(xv, wci[...], preferred_element_type=jnp.float32)
                       + jnp.dot(out, wco[...], preferred_element_type=jnp.float32)
                       + bc[...])
    return z * out + (1.0 - z) * xv


def _dense_mid(part, h, x, deg, wt, wb, b, wci, wco, bc, o):
    o[...] = jnp.maximum(_gated(part, h, x, deg, wt, wb, b, wci, wco, bc), 0.0)


def _dense_fin(part, h, x, deg, wt, wb, b, wci, wco, bc, wfc, bfc, o):
    g = _gated(part, h, x, deg, wt, wb, b, wci, wco, bc)
    o[...] = jnp.dot(g, wfc[...], preferred_element_type=jnp.float32) + bfc[...]


def _row_spec():
    return pl.BlockSpec((BLK, F), lambda i: (i, 0))


def _full_spec(shape):
    nd = len(shape)
    return pl.BlockSpec(shape, lambda i: (0,) * nd)


def _dense_call(final, part, h, x, degp, wt, wb, b, wci, wco, bc, *fc):
    in_specs = [
        pl.BlockSpec((NC, BLK, F), lambda i: (0, i, 0)),
        _row_spec(),
        _row_spec(),
        pl.BlockSpec((NC, BLK, DW), lambda i: (0, i, 0)),
        _full_spec((F, F)), _full_spec((F, F)),
        _full_spec((1, F)),
        _full_spec((F, F)), _full_spec((F, F)),
        _full_spec((1, F)),
    ]
    args = [part, h, x, degp, wt, wb, b, wci, wco, bc]
    if final:
        in_specs += [_full_spec((F, F)), _full_spec((1, F))]
        args += list(fc)
    return pl.pallas_call(
        _dense_fin if final else _dense_mid,
        grid=(N // BLK,),
        in_specs=in_specs,
        out_specs=_row_spec(),
        out_shape=jax.ShapeDtypeStruct((N, F), jnp.float32),
    )(*args)


def kernel(x, edge_index, W0, b0, W1, b1, W2, b2, Wci, bci, Wco, bco, Wfc, bfc):
    row = edge_index[0]
    col = edge_index[1]

    pad = E_PAD - E
    pad_i = jnp.arange(pad, dtype=jnp.int32)
    row3 = jnp.concatenate([row, N + pad_i % (ACC_ROWS - N)]).reshape(
        NW, NCHUNK, CH)
    col3 = jnp.concatenate([col, pad_i % 64]).reshape(NW, NCHUNK, CH)

    segsum_deg, segsum = _sc_kernels()

    bc = (bci + bco).reshape(1, F)
    bfc2 = bfc.reshape(1, F)

    part, degp = segsum_deg(x, col3, row3)
    h1 = _dense_call(False, part, x, x, degp, W0[:F], W0[F:], b0,
                     Wci, Wco, bc)
    part, _ = segsum(h1, col3, row3)
    h2 = _dense_call(False, part, h1, x, degp, W1[:F], W1[F:], b1,
                     Wci, Wco, bc)
    part, _ = segsum(h2, col3, row3)
    return _dense_call(True, part, h2, x, degp, W2[:F], W2[F:], b2,
                       Wci, Wco, bc, Wfc, bfc2)


# speedup vs baseline: 9.3079x; 1.1088x over previous
import functools

import jax
import jax.numpy as jnp
from jax import lax
from jax.experimental import pallas as pl
from jax.experimental.pallas import tpu as pltpu
from jax.experimental.pallas import tpu_sc as plsc

N = 10000
E = 320000
F = 128

NC = 2
NS = 16
NW = NC * NS
L = 16
DW = 16

CH = 64
CH2 = 128
GCH = 8
NCHUNK = 160
EPW = CH * NCHUNK
E_PAD = EPW * NW
ACC_ROWS = 10240
RPT_Z = ACC_ROWS // NS


def _make_segsum_body(with_deg, ch):
    nchunk = E_PAD // (NW * ch)
    ngrp = nchunk // GCH
    zch = RPT_Z // ch
    def body_fn(*refs):
        if with_deg:
            (h_hbm, col_hbm, row_hbm, out_hbm, dout_hbm,
             colb, rowb, rows0, rows1, idxb, ob, zdb, acc, dacc,
             gsem0, gsem1, ssem0, ssem1, sem) = refs
        else:
            (h_hbm, col_hbm, row_hbm, out_hbm, dout_hbm,
             colb, rowb, rows0, rows1, idxb, acc,
             gsem0, gsem1, ssem0, ssem1, sem) = refs
        rowsb = (rows0, rows1)
        gsem = (gsem0, gsem1)
        ssem = (ssem0, ssem1)
        rows = rows0
        c = lax.axis_index("c")
        s = lax.axis_index("s")
        w = c * NS + s

        zv = jnp.zeros((L,), jnp.float32)

        def zrow(i, carry):
            for j in range(F // L):
                rows[i, pl.ds(j * L, L)] = zv
            return carry
        lax.fori_loop(0, ch, zrow, 0)
        if with_deg:
            ov = jnp.ones((L,), jnp.float32)
            for i in range(ch):
                ob[i, :] = ov
                zdb[i, :] = zv

        iv = lax.iota(jnp.int32, L)

        def fill_idx(base):
            for j in range(ch // L):
                idxb[pl.ds(j * L, L)] = iv + (base + j * L)

        def zloop(k, carry):
            fill_idx(s * RPT_Z + k * ch)
            pltpu.sync_copy(rows, acc.at[idxb])
            if with_deg:
                pltpu.sync_copy(zdb, dacc.at[idxb])
            return carry

        lax.fori_loop(0, zch, zloop, 0)
        plsc.subcore_barrier()

        pltpu.sync_copy(col_hbm.at[w, pl.ds(0, GCH)], colb)
        pltpu.sync_copy(row_hbm.at[w, pl.ds(0, GCH)], rowb)
        pltpu.async_copy(h_hbm.at[colb.at[0]], rows0, gsem0)

        def grp_loop(grp, carry):
            for j in range(GCH):
                b = j & 1
                nb = 1 - b
                if j < GCH - 1:
                    if j > 0:
                        pltpu.make_async_copy(rowsb[nb],
                                              acc.at[rowb.at[j]],
                                              ssem[nb]).wait()
                    pltpu.async_copy(h_hbm.at[colb.at[j + 1]], rowsb[nb],
                                     gsem[nb])
                    pltpu.make_async_copy(h_hbm.at[colb.at[j]], rowsb[b],
                                          gsem[b]).wait()
                    pltpu.async_copy(rowsb[b], acc.at[rowb.at[j]], ssem[b],
                                     add=True)
                    if with_deg:
                        pltpu.sync_copy(ob, dacc.at[rowb.at[j]], add=True)
                else:
                    pltpu.make_async_copy(rowsb[nb], acc.at[rowb.at[j]],
                                          ssem[nb]).wait()
                    pltpu.make_async_copy(h_hbm.at[colb.at[j]], rowsb[b],
                                          gsem[b]).wait()
                    pltpu.sync_copy(rowsb[b], acc.at[rowb.at[j]], add=True)
                    if with_deg:
                        pltpu.sync_copy(ob, dacc.at[rowb.at[j]], add=True)

                    @pl.when(grp + 1 < ngrp)
                    def _():
                        pltpu.sync_copy(
                            col_hbm.at[w, pl.ds((grp + 1) * GCH, GCH)], colb)
                        pltpu.sync_copy(
                            row_hbm.at[w, pl.ds((grp + 1) * GCH, GCH)], rowb)
                        pltpu.async_copy(h_hbm.at[colb.at[0]], rowsb[nb],
                                         gsem[nb])
            return carry

        lax.fori_loop(0, ngrp, grp_loop, 0)

        plsc.subcore_barrier()

        def oloop(k, carry):
            base = s * RPT_Z + k * ch
            fill_idx(base)
            pltpu.async_copy(acc.at[idxb], rows, sem).wait()
            pltpu.sync_copy(rows, out_hbm.at[c, pl.ds(base, ch), :])
            if with_deg:
                pltpu.async_copy(dacc.at[idxb], zdb, sem).wait()
                pltpu.sync_copy(zdb, dout_hbm.at[c, pl.ds(base, ch), :])
            return carry

        lax.fori_loop(0, zch, oloop, 0)
    return body_fn


@functools.cache
def _sc_kernels():
    mesh = plsc.VectorSubcoreMesh(core_axis_name="c", subcore_axis_name="s")

    def make(with_deg, ch):
        deg_scratch = [
            pltpu.VMEM((ch, DW), jnp.float32),
            pltpu.VMEM((ch, DW), jnp.float32),
        ] if with_deg else []
        deg_accum = [
            pltpu.VMEM_SHARED((ACC_ROWS, DW), jnp.float32),
        ] if with_deg else []
        dout_rows = ACC_ROWS if with_deg else 16
        return pl.kernel(
            _make_segsum_body(with_deg, ch),
            mesh=mesh,
            out_type=(jax.ShapeDtypeStruct((NC, ACC_ROWS, F), jnp.float32),
                      jax.ShapeDtypeStruct((NC, dout_rows, DW), jnp.float32)),
            scratch_types=[
                pltpu.VMEM((GCH, ch), jnp.int32),
                pltpu.VMEM((GCH, ch), jnp.int32),
                pltpu.VMEM((ch, F), jnp.float32),
                pltpu.VMEM((ch, F), jnp.float32),
                pltpu.VMEM((ch,), jnp.int32),
                *deg_scratch,
                pltpu.VMEM_SHARED((ACC_ROWS, F), jnp.float32),
                *deg_accum,
                pltpu.SemaphoreType.DMA,
                pltpu.SemaphoreType.DMA,
                pltpu.SemaphoreType.DMA,
                pltpu.SemaphoreType.DMA,
                pltpu.SemaphoreType.DMA,
            ],
        )

    return make(True, CH), make(False, CH2)


BLK = 1000


def _gated(part, h, x, deg, wt, wb, b, wci, wco, bc):
    sup = (part[0] + part[1]) / (deg[0, :, 0:1] + deg[1, :, 0:1])
    hv = h[...]
    xv = x[...]
    out = (jnp.dot(hv, wt[...], preferred_element_type=jnp.float32)
           + jnp.dot(sup, wb[...], preferred_element_type=jnp.float32)
           + b[...])
    z = jax.nn.sigmoid(jnp.dot(xv, wci[...], preferred_element_type=jnp.float32)
                       + jnp.dot(out, wco[...], preferred_element_type=jnp.float32)
                       + bc[...])
    return z * out + (1.0 - z) * xv


def _dense_mid(part, h, x, deg, wt, wb, b, wci, wco, bc, o):
    o[...] = jnp.maximum(_gated(part, h, x, deg, wt, wb, b, wci, wco, bc), 0.0)


def _dense_fin(part, h, x, deg, wt, wb, b, wci, wco, bc, wfc, bfc, o):
    g = _gated(part, h, x, deg, wt, wb, b, wci, wco, bc)
    o[...] = jnp.dot(g, wfc[...], preferred_element_type=jnp.float32) + bfc[...]


def _row_spec():
    return pl.BlockSpec((BLK, F), lambda i: (i, 0))


def _full_spec(shape):
    nd = len(shape)
    return pl.BlockSpec(shape, lambda i: (0,) * nd)


def _dense_call(final, part, h, x, degp, wt, wb, b, wci, wco, bc, *fc):
    in_specs = [
        pl.BlockSpec((NC, BLK, F), lambda i: (0, i, 0)),
        _row_spec(),
        _row_spec(),
        pl.BlockSpec((NC, BLK, DW), lambda i: (0, i, 0)),
        _full_spec((F, F)), _full_spec((F, F)),
        _full_spec((1, F)),
        _full_spec((F, F)), _full_spec((F, F)),
        _full_spec((1, F)),
    ]
    args = [part, h, x, degp, wt, wb, b, wci, wco, bc]
    if final:
        in_specs += [_full_spec((F, F)), _full_spec((1, F))]
        args += list(fc)
    return pl.pallas_call(
        _dense_fin if final else _dense_mid,
        grid=(N // BLK,),
        in_specs=in_specs,
        out_specs=_row_spec(),
        out_shape=jax.ShapeDtypeStruct((N, F), jnp.float32),
    )(*args)


def kernel(x, edge_index, W0, b0, W1, b1, W2, b2, Wci, bci, Wco, bco, Wfc, bfc):
    row = edge_index[0]
    col = edge_index[1]

    pad = E_PAD - E
    pad_i = jnp.arange(pad, dtype=jnp.int32)
    rowp = jnp.concatenate([row, N + pad_i % (ACC_ROWS - N)])
    colp = jnp.concatenate([col, pad_i % 64])
    row3 = rowp.reshape(NW, NCHUNK, CH)
    col3 = colp.reshape(NW, NCHUNK, CH)
    row3b = rowp.reshape(NW, EPW // CH2, CH2)
    col3b = colp.reshape(NW, EPW // CH2, CH2)

    segsum_deg, segsum = _sc_kernels()

    bc = (bci + bco).reshape(1, F)
    bfc2 = bfc.reshape(1, F)

    part, degp = segsum_deg(x, col3, row3)
    h1 = _dense_call(False, part, x, x, degp, W0[:F], W0[F:], b0,
                     Wci, Wco, bc)
    part, _ = segsum(h1, col3b, row3b)
    h2 = _dense_call(False, part, h1, x, degp, W1[:F], W1[F:], b1,
                     Wci, Wco, bc)
    part, _ = segsum(h2, col3b, row3b)
    return _dense_call(True, part, h2, x, degp, W2[:F], W2[F:], b2,
                       Wci, Wco, bc, Wfc, bfc2)


# speedup vs baseline: 9.4832x; 1.0188x over previous
import functools

import jax
import jax.numpy as jnp
from jax import lax
from jax.experimental import pallas as pl
from jax.experimental.pallas import tpu as pltpu
from jax.experimental.pallas import tpu_sc as plsc

N = 10000
E = 320000
F = 128

NC = 2
NS = 16
NW = NC * NS
L = 16
DW = 16

CH = 64
CH2 = 128
GCH = 8
NCHUNK = 160
EPW = CH * NCHUNK
E_PAD = EPW * NW
ACC_ROWS = 10240
RPT_Z = ACC_ROWS // NS


def _make_segsum_body(with_deg, ch):
    nchunk = E_PAD // (NW * ch)
    ngrp = nchunk // GCH
    zch = RPT_Z // ch
    def body_fn(*refs):
        if with_deg:
            (h_hbm, col_hbm, row_hbm, out_hbm, dout_hbm,
             colb, rowb, rows0, rows1, idxb, ob, zdb, acc, dacc,
             gsem0, gsem1, ssem0, ssem1, sem, dsem) = refs
        else:
            (h_hbm, col_hbm, row_hbm, out_hbm, dout_hbm,
             colb, rowb, rows0, rows1, idxb, acc,
             gsem0, gsem1, ssem0, ssem1, sem) = refs
        rowsb = (rows0, rows1)
        gsem = (gsem0, gsem1)
        ssem = (ssem0, ssem1)
        rows = rows0
        c = lax.axis_index("c")
        s = lax.axis_index("s")
        w = c * NS + s

        zv = jnp.zeros((L,), jnp.float32)

        def zrow(i, carry):
            for j in range(F // L):
                rows[i, pl.ds(j * L, L)] = zv
            return carry
        lax.fori_loop(0, ch, zrow, 0)
        if with_deg:
            ov = jnp.ones((L,), jnp.float32)
            for i in range(ch):
                ob[i, :] = ov
                zdb[i, :] = zv

        iv = lax.iota(jnp.int32, L)

        def fill_idx(base):
            for j in range(ch // L):
                idxb[pl.ds(j * L, L)] = iv + (base + j * L)

        def zloop(k, carry):
            fill_idx(s * RPT_Z + k * ch)
            pltpu.sync_copy(rows, acc.at[idxb])
            if with_deg:
                pltpu.sync_copy(zdb, dacc.at[idxb])
            return carry

        lax.fori_loop(0, zch, zloop, 0)
        plsc.subcore_barrier()

        pltpu.sync_copy(col_hbm.at[w, pl.ds(0, GCH)], colb)
        pltpu.sync_copy(row_hbm.at[w, pl.ds(0, GCH)], rowb)
        pltpu.async_copy(h_hbm.at[colb.at[0]], rows0, gsem0)

        def grp_loop(grp, carry):
            for j in range(GCH):
                b = j & 1
                nb = 1 - b
                if j < GCH - 1:
                    if j > 0:
                        pltpu.make_async_copy(rowsb[nb],
                                              acc.at[rowb.at[j]],
                                              ssem[nb]).wait()
                    pltpu.async_copy(h_hbm.at[colb.at[j + 1]], rowsb[nb],
                                     gsem[nb])
                    pltpu.make_async_copy(h_hbm.at[colb.at[j]], rowsb[b],
                                          gsem[b]).wait()
                    pltpu.async_copy(rowsb[b], acc.at[rowb.at[j]], ssem[b],
                                     add=True)
                    if with_deg:
                        pltpu.async_copy(ob, dacc.at[rowb.at[j]], dsem,
                                         add=True)
                else:
                    pltpu.make_async_copy(rowsb[nb], acc.at[rowb.at[j]],
                                          ssem[nb]).wait()
                    pltpu.make_async_copy(h_hbm.at[colb.at[j]], rowsb[b],
                                          gsem[b]).wait()
                    pltpu.sync_copy(rowsb[b], acc.at[rowb.at[j]], add=True)
                    if with_deg:
                        pltpu.sync_copy(ob, dacc.at[rowb.at[j]], add=True)
                        for _ in range(GCH - 1):
                            pltpu.make_async_copy(ob, dacc.at[rowb.at[j]],
                                                  dsem).wait()

                    @pl.when(grp + 1 < ngrp)
                    def _():
                        pltpu.sync_copy(
                            col_hbm.at[w, pl.ds((grp + 1) * GCH, GCH)], colb)
                        pltpu.sync_copy(
                            row_hbm.at[w, pl.ds((grp + 1) * GCH, GCH)], rowb)
                        pltpu.async_copy(h_hbm.at[colb.at[0]], rowsb[nb],
                                         gsem[nb])
            return carry

        lax.fori_loop(0, ngrp, grp_loop, 0)

        plsc.subcore_barrier()

        def oloop(k, carry):
            base = s * RPT_Z + k * ch
            fill_idx(base)
            pltpu.async_copy(acc.at[idxb], rows, sem).wait()
            pltpu.sync_copy(rows, out_hbm.at[c, pl.ds(base, ch), :])
            if with_deg:
                pltpu.async_copy(dacc.at[idxb], zdb, sem).wait()
                pltpu.sync_copy(zdb, dout_hbm.at[c, pl.ds(base, ch), :])
            return carry

        lax.fori_loop(0, zch, oloop, 0)
    return body_fn


@functools.cache
def _sc_kernels():
    mesh = plsc.VectorSubcoreMesh(core_axis_name="c", subcore_axis_name="s")

    def make(with_deg, ch):
        deg_scratch = [
            pltpu.VMEM((ch, DW), jnp.float32),
            pltpu.VMEM((ch, DW), jnp.float32),
        ] if with_deg else []
        deg_accum = [
            pltpu.VMEM_SHARED((ACC_ROWS, DW), jnp.float32),
        ] if with_deg else []
        dout_rows = ACC_ROWS if with_deg else 16
        return pl.kernel(
            _make_segsum_body(with_deg, ch),
            mesh=mesh,
            out_type=(jax.ShapeDtypeStruct((NC, ACC_ROWS, F), jnp.float32),
                      jax.ShapeDtypeStruct((NC, dout_rows, DW), jnp.float32)),
            scratch_types=[
                pltpu.VMEM((GCH, ch), jnp.int32),
                pltpu.VMEM((GCH, ch), jnp.int32),
                pltpu.VMEM((ch, F), jnp.float32),
                pltpu.VMEM((ch, F), jnp.float32),
                pltpu.VMEM((ch,), jnp.int32),
                *deg_scratch,
                pltpu.VMEM_SHARED((ACC_ROWS, F), jnp.float32),
                *deg_accum,
                pltpu.SemaphoreType.DMA,
                pltpu.SemaphoreType.DMA,
                pltpu.SemaphoreType.DMA,
                pltpu.SemaphoreType.DMA,
                pltpu.SemaphoreType.DMA,
                *([pltpu.SemaphoreType.DMA] if with_deg else []),
            ],
        )

    return make(True, CH), make(False, CH2)


BLK = 1000


def _gated(part, h, x, deg, wt, wb, b, wci, wco, bc):
    sup = (part[0] + part[1]) / (deg[0, :, 0:1] + deg[1, :, 0:1])
    hv = h[...]
    xv = x[...]
    out = (jnp.dot(hv, wt[...], preferred_element_type=jnp.float32)
           + jnp.dot(sup, wb[...], preferred_element_type=jnp.float32)
           + b[...])
    z = jax.nn.sigmoid(jnp.dot(xv, wci[...], preferred_element_type=jnp.float32)
                       + jnp.dot(out, wco[...], preferred_element_type=jnp.float32)
                       + bc[...])
    return z * out + (1.0 - z) * xv


def _dense_mid(part, h, x, deg, wt, wb, b, wci, wco, bc, o):
    o[...] = jnp.maximum(_gated(part, h, x, deg, wt, wb, b, wci, wco, bc), 0.0)


def _dense_fin(part, h, x, deg, wt, wb, b, wci, wco, bc, wfc, bfc, o):
    g = _gated(part, h, x, deg, wt, wb, b, wci, wco, bc)
    o[...] = jnp.dot(g, wfc[...], preferred_element_type=jnp.float32) + bfc[...]


def _row_spec():
    return pl.BlockSpec((BLK, F), lambda i: (i, 0))


def _full_spec(shape):
    nd = len(shape)
    return pl.BlockSpec(shape, lambda i: (0,) * nd)


def _dense_call(final, part, h, x, degp, wt, wb, b, wci, wco, bc, *fc):
    in_specs = [
        pl.BlockSpec((NC, BLK, F), lambda i: (0, i, 0)),
        _row_spec(),
        _row_spec(),
        pl.BlockSpec((NC, BLK, DW), lambda i: (0, i, 0)),
        _full_spec((F, F)), _full_spec((F, F)),
        _full_spec((1, F)),
        _full_spec((F, F)), _full_spec((F, F)),
        _full_spec((1, F)),
    ]
    args = [part, h, x, degp, wt, wb, b, wci, wco, bc]
    if final:
        in_specs += [_full_spec((F, F)), _full_spec((1, F))]
        args += list(fc)
    return pl.pallas_call(
        _dense_fin if final else _dense_mid,
        grid=(N // BLK,),
        in_specs=in_specs,
        out_specs=_row_spec(),
        out_shape=jax.ShapeDtypeStruct((N, F), jnp.float32),
    )(*args)


def kernel(x, edge_index, W0, b0, W1, b1, W2, b2, Wci, bci, Wco, bco, Wfc, bfc):
    row = edge_index[0]
    col = edge_index[1]

    pad = E_PAD - E
    pad_i = jnp.arange(pad, dtype=jnp.int32)
    rowp = jnp.concatenate([row, N + pad_i % (ACC_ROWS - N)])
    colp = jnp.concatenate([col, pad_i % 64])
    row3 = rowp.reshape(NW, NCHUNK, CH)
    col3 = colp.reshape(NW, NCHUNK, CH)
    row3b = rowp.reshape(NW, EPW // CH2, CH2)
    col3b = colp.reshape(NW, EPW // CH2, CH2)

    segsum_deg, segsum = _sc_kernels()

    bc = (bci + bco).reshape(1, F)
    bfc2 = bfc.reshape(1, F)

    part, degp = segsum_deg(x, col3, row3)
    h1 = _dense_call(False, part, x, x, degp, W0[:F], W0[F:], b0,
                     Wci, Wco, bc)
    part, _ = segsum(h1, col3b, row3b)
    h2 = _dense_call(False, part, h1, x, degp, W1[:F], W1[F:], b1,
                     Wci, Wco, bc)
    part, _ = segsum(h2, col3b, row3b)
    return _dense_call(True, part, h2, x, degp, W2[:F], W2[F:], b2,
                       Wci, Wco, bc, Wfc, bfc2)
